# baseline (speedup 1.0000x reference)
import functools

import jax
import jax.numpy as jnp
from jax import lax
from jax.experimental import pallas as pl
from jax.experimental.pallas import tpu as pltpu
from jax.experimental.pallas import tpu_sc as plsc

_NC = 2
_NS = 16


def kernel(hidden_states, position_embeddings):
    B, S, D = hidden_states.shape
    assert position_embeddings.shape == (S, D)
    NW = _NC * _NS
    rows_per_w = S // NW
    CH = 32
    n_ch = rows_per_w // CH
    mesh = plsc.VectorSubcoreMesh(core_axis_name="c", subcore_axis_name="s")

    @functools.partial(
        pl.kernel,
        mesh=mesh,
        out_type=jax.ShapeDtypeStruct((B, S, D), jnp.float32),
        scratch_types=[
            pltpu.VMEM((CH, D), jnp.float32),
            pltpu.SemaphoreType.DMA,
        ],
    )
    def sc_bcast(table_hbm, out_hbm, buf, sem):
        wid = lax.axis_index("s") * _NC + lax.axis_index("c")
        base = wid * rows_per_w

        def body(i, carry):
            r0 = base + i * CH
            pltpu.sync_copy(table_hbm.at[pl.ds(r0, CH)], buf)
            for b in range(B):
                pltpu.sync_copy(buf, out_hbm.at[b, pl.ds(r0, CH)])
            return carry

        lax.fori_loop(0, n_ch, body, 0)

    return sc_bcast(position_embeddings)


# speedup vs baseline: 1.2990x; 1.2990x over previous
import jax
import jax.numpy as jnp
from jax.experimental import pallas as pl
from jax.experimental.pallas import tpu as pltpu

_BLK_S = 512


def _fanout(table_ref, out_any, sem):
    j = pl.program_id(0)
    B = out_any.shape[0]
    cps = [
        pltpu.make_async_copy(
            table_ref, out_any.at[b, pl.ds(j * _BLK_S, _BLK_S), :], sem.at[b]
        )
        for b in range(B)
    ]
    for c in cps:
        c.start()
    for c in cps:
        c.wait()


def kernel(hidden_states, position_embeddings):
    B, S, D = hidden_states.shape
    assert position_embeddings.shape == (S, D)
    grid = (S // _BLK_S,)
    return pl.pallas_call(
        _fanout,
        grid=grid,
        in_specs=[pl.BlockSpec((_BLK_S, D), lambda j: (j, 0))],
        out_specs=pl.BlockSpec(memory_space=pl.ANY),
        out_shape=jax.ShapeDtypeStruct((B, S, D), position_embeddings.dtype),
        scratch_shapes=[pltpu.SemaphoreType.DMA((4,))],
    )(position_embeddings)


# speedup vs baseline: 1.5184x; 1.1689x over previous
import jax
import jax.numpy as jnp
from jax.experimental import pallas as pl


def _bcast_copy(table_ref, out_ref):
    blk = table_ref[...]
    out_ref[...] = jnp.broadcast_to(blk[None, :, :], out_ref.shape)


def kernel(hidden_states, position_embeddings):
    B, S, D = hidden_states.shape
    assert position_embeddings.shape == (S, D)
    BLK_S = 1024
    grid = (S // BLK_S,)
    return pl.pallas_call(
        _bcast_copy,
        grid=grid,
        in_specs=[pl.BlockSpec((BLK_S, D), lambda j: (j, 0))],
        out_specs=pl.BlockSpec((B, BLK_S, D), lambda j: (0, j, 0)),
        out_shape=jax.ShapeDtypeStruct((B, S, D), position_embeddings.dtype),
    )(position_embeddings)
